# baseline (speedup 1.0000x reference)
import dataclasses
import functools

import jax
import jax.numpy as jnp
from jax import lax
from jax.experimental import pallas as pl
from jax.experimental.pallas import tpu as pltpu
from jax.experimental.pallas import tpu_sc as plsc

_N = 10000
_E = 320000
_D = 128
_C = 16
_BLK = 1000

_NC = 2
_NS = 16
_NW = _NC * _NS
_EPT = _E // _NW
_CH = 125
_NCH = _EPT // _CH
_STRIPE = 632
_STRIPE_LAST = _N - _STRIPE * (_NS - 1)


def _seg_sum_body(width, with_deg, table_hbm, src_hbm, dst_hbm, zeros_hbm,
                  out_hbm, deg_hbm, src_v, dst_v, rows_v, ones_v, zdeg_v,
                  acc_sh, deg_sh):
    cid = lax.axis_index("c")
    sid = lax.axis_index("s")
    wid = sid * _NC + cid

    base = sid * _STRIPE

    @pl.when(sid < _NS - 1)
    def _():
        pltpu.sync_copy(zeros_hbm.at[pl.ds(base, _STRIPE)],
                        acc_sh.at[pl.ds(base, _STRIPE)])

    @pl.when(sid == _NS - 1)
    def _():
        pltpu.sync_copy(zeros_hbm.at[pl.ds(base, _STRIPE_LAST)],
                        acc_sh.at[pl.ds(base, _STRIPE_LAST)])

    if with_deg:
        @pl.when(sid == 0)
        def _():
            @pl.loop(0, _N // 16)
            def _(i):
                zdeg_v[pl.ds(i * 16, 16)] = jnp.zeros((16,), jnp.float32)
            pltpu.sync_copy(zdeg_v, deg_sh)
        for k in range(8):
            ones_v[pl.ds(k * 16, 16)] = jnp.ones((16,), jnp.float32)

    plsc.subcore_barrier()

    pltpu.sync_copy(src_hbm.at[wid], src_v)
    pltpu.sync_copy(dst_hbm.at[wid], dst_v)

    @pl.loop(0, _NCH)
    def _(j):
        pltpu.sync_copy(table_hbm.at[src_v.at[j]], rows_v)
        pltpu.sync_copy(rows_v, acc_sh.at[dst_v.at[j]], add=True)
        if with_deg:
            pltpu.sync_copy(ones_v.at[pl.ds(0, _CH)],
                            deg_sh.at[dst_v.at[j]], add=True)

    plsc.subcore_barrier()

    @pl.when(sid < _NS - 1)
    def _():
        pltpu.sync_copy(acc_sh.at[pl.ds(base, _STRIPE)],
                        out_hbm.at[cid, pl.ds(base, _STRIPE)])

    @pl.when(sid == _NS - 1)
    def _():
        pltpu.sync_copy(acc_sh.at[pl.ds(base, _STRIPE_LAST)],
                        out_hbm.at[cid, pl.ds(base, _STRIPE_LAST)])
    if with_deg:
        @pl.when(sid == 0)
        def _():
            pltpu.sync_copy(deg_sh, deg_hbm.at[cid])


def _seg_sum_sc(table, src3d, dst3d, width, with_deg):
    mesh = plsc.VectorSubcoreMesh(core_axis_name="c", subcore_axis_name="s")
    out_type = [jax.ShapeDtypeStruct((_NC, _N, width), jnp.float32)]
    if with_deg:
        out_type.append(jax.ShapeDtypeStruct((_NC, _N), jnp.float32))
    scratch = [
        pltpu.VMEM((_NCH, _CH), jnp.int32),
        pltpu.VMEM((_NCH, _CH), jnp.int32),
        pltpu.VMEM((_CH, width), jnp.float32),
        pltpu.VMEM((_CH + 3,), jnp.float32),
        pltpu.VMEM((_N,), jnp.float32),
        pltpu.VMEM_SHARED((_N, width), jnp.float32),
        pltpu.VMEM_SHARED((_N,), jnp.float32),
    ]
    zeros = jnp.zeros((_N, width), jnp.float32)
    body = functools.partial(_seg_sum_body, width, with_deg)
    if not with_deg:
        def body2(table_hbm, src_hbm, dst_hbm, zeros_hbm, out_hbm, *rest):
            return functools.partial(_seg_sum_body, width, False)(
                table_hbm, src_hbm, dst_hbm, zeros_hbm, out_hbm, None, *rest)
        fn = pl.kernel(body2, out_type=out_type, mesh=mesh,
                       scratch_types=scratch)
        return fn(table, src3d, dst3d, zeros)[0], None
    fn = pl.kernel(body, out_type=out_type, mesh=mesh, scratch_types=scratch)
    outs = fn(table, src3d, dst3d, zeros)
    return outs[0], outs[1]


_MROWS = 632
_MROWS_LAST = _N - _MROWS * (_NS - 1)
_EH = _E // _NC
_SCH = 2000
_NSCH = _EH // _SCH
_GCH = 128
_MCAP = _SCH + 2 * _GCH
_DUMP = _MROWS


def _seg_max_body(hp_hbm, src_hbm, dst_hbm, zeros_hbm, out_hbm,
                  sbuf, dbuf, msrc, mdst, gidx, rows_v, acc_v):
    cid = lax.axis_index("c")
    sid = lax.axis_index("s")
    base = sid * _MROWS
    size = jnp.where(sid < _NS - 1, _MROWS, _MROWS_LAST)

    @pl.when(sid < _NS - 1)
    def _():
        pltpu.sync_copy(zeros_hbm.at[pl.ds(base, _MROWS)],
                        acc_v.at[pl.ds(0, _MROWS)])

    @pl.when(sid == _NS - 1)
    def _():
        pltpu.sync_copy(zeros_hbm.at[pl.ds(base, _MROWS_LAST)],
                        acc_v.at[pl.ds(0, _MROWS_LAST)])

    ebase = cid * _EH
    zeros16i = jnp.zeros((16,), jnp.int32)
    dump16i = jnp.full((16,), _DUMP, jnp.int32)
    full_mask = jnp.ones((16,), jnp.bool_)

    @pl.loop(0, _NSCH)
    def _(ci):
        off = ebase + ci * _SCH
        pltpu.sync_copy(src_hbm.at[pl.ds(off, _SCH)], sbuf.at[pl.ds(0, _SCH)])
        pltpu.sync_copy(dst_hbm.at[pl.ds(off, _SCH)], dbuf.at[pl.ds(0, _SCH)])

        def scan_g(g, ptr):
            d16 = dbuf[pl.ds(g * 16, 16)]
            t16 = d16 - base
            mask = (d16 >= base) & (t16 < size)
            s16 = sbuf[pl.ds(g * 16, 16)]
            plsc.store_compressed(msrc.at[pl.ds(ptr, 16)], s16, mask=mask)
            plsc.store_compressed(mdst.at[pl.ds(ptr, 16)], t16, mask=mask)
            pc = plsc.all_reduce_population_count(mask)
            pc = pc if pc.ndim == 0 else pc[0]
            return ptr + pc

        nm = lax.fori_loop(0, _SCH // 16, scan_g, jnp.int32(0))

        for k in range(_GCH // 16):
            plsc.store_compressed(msrc.at[pl.ds(nm + k * 16, 16)],
                                  zeros16i, mask=full_mask)
            plsc.store_compressed(mdst.at[pl.ds(nm + k * 16, 16)],
                                  dump16i, mask=full_mask)

        @pl.loop(0, _SCH // _GCH)
        def _(g):
            @pl.when(g * _GCH < nm)
            def _():
                goff = pl.multiple_of(g * _GCH, _GCH)
                for k in range(_GCH // 16):
                    gidx[0, pl.ds(k * 16, 16)] = msrc[pl.ds(goff + k * 16, 16)]
                pltpu.sync_copy(hp_hbm.at[gidx.at[0]], rows_v)

                def accum_grp(gg, _):
                    joff = pl.multiple_of(g * _GCH + gg * 16, 16)
                    ld16 = mdst[pl.ds(joff, 16)]
                    for l in range(16):
                        ld = ld16[l]
                        j = gg * 16 + l
                        for k in range(8):
                            sl = pl.ds(k * 16, 16)
                            acc_v[ld, sl] = jnp.maximum(acc_v[ld, sl],
                                                        rows_v[j, sl])
                    return 0

                lax.fori_loop(0, _GCH // 16, accum_grp, 0)

    @pl.when(sid < _NS - 1)
    def _():
        pltpu.sync_copy(acc_v.at[pl.ds(0, _MROWS)],
                        out_hbm.at[cid, pl.ds(base, _MROWS)])

    @pl.when(sid == _NS - 1)
    def _():
        pltpu.sync_copy(acc_v.at[pl.ds(0, _MROWS_LAST)],
                        out_hbm.at[cid, pl.ds(base, _MROWS_LAST)])


def _seg_max_sc(hp, src, dst):
    mesh = plsc.VectorSubcoreMesh(core_axis_name="c", subcore_axis_name="s")
    scratch = [
        pltpu.VMEM((_SCH,), jnp.int32),
        pltpu.VMEM((_SCH,), jnp.int32),
        pltpu.VMEM((_MCAP,), jnp.int32),
        pltpu.VMEM((_MCAP,), jnp.int32),
        pltpu.VMEM((1, _GCH), jnp.int32),
        pltpu.VMEM((_GCH, _D), jnp.float32),
        pltpu.VMEM((_MROWS + 8, _D), jnp.float32),
    ]
    zeros = jnp.zeros((_N, _D), jnp.float32)
    cp = pltpu.CompilerParams()
    if "needs_layout_passes" in pltpu.CompilerParams.__dataclass_fields__:
        cp = dataclasses.replace(cp, needs_layout_passes=False)
    fn = pl.kernel(_seg_max_body,
                   out_type=jax.ShapeDtypeStruct((_NC, _N, _D), jnp.float32),
                   mesh=mesh, scratch_types=scratch, compiler_params=cp)
    return fn(hp, src, dst, zeros)


def _dense_a_body(x_ref, wp_ref, bp_ref, wm_ref, wq_ref, hp_ref, xm_ref, xp_ref):
    xb = x_ref[...]
    hp_ref[...] = jnp.maximum(
        jnp.dot(xb, wp_ref[...], preferred_element_type=jnp.float32)
        + bp_ref[...], 0.0)
    xm_ref[...] = jnp.dot(xb, wm_ref[...], preferred_element_type=jnp.float32)
    xp_ref[...] = jnp.dot(xb, wq_ref[...], preferred_element_type=jnp.float32)


def _dense_a(x, W_pool, b_pool, W_self_m, W_self_p):
    n = x.shape[0]
    grid = (n // _BLK,)
    blk = pl.BlockSpec((_BLK, _D), lambda i: (i, 0))
    wspec = pl.BlockSpec((_D, _D), lambda i: (0, 0))
    bspec = pl.BlockSpec((1, _D), lambda i: (0, 0))
    out_sd = jax.ShapeDtypeStruct((n, _D), jnp.float32)
    return pl.pallas_call(
        _dense_a_body,
        grid=grid,
        in_specs=[blk, wspec, bspec, wspec, wspec],
        out_specs=[blk, blk, blk],
        out_shape=[out_sd, out_sd, out_sd],
    )(x, W_pool, b_pool.reshape(1, _D), W_self_m, W_self_p)


def kernel(x, edge_index, W_self_m, W_neigh_m, b_m, W_pool, b_pool,
           W_self_p, W_neigh_p, b_p, gamma_m, beta_m, gamma_p, beta_p,
           W_self_o, W_neigh_o, b_o):
    src3d = edge_index[0].reshape(_NW, _NCH, _CH)
    dst3d = edge_index[1].reshape(_NW, _NCH, _CH)

    hp, xm, xp = _dense_a(x, W_pool, b_pool, W_self_m, W_self_p)

    sumx_p, deg_p = _seg_sum_sc(x, src3d, dst3d, _D, True)
    deg = jnp.maximum(deg_p[0] + deg_p[1], 1.0)
    aggx = (sumx_p[0] + sumx_p[1]) / deg[:, None]
    m = jax.nn.relu(xm + aggx @ W_neigh_m + b_m)

    mx_p = _seg_max_sc(hp, edge_index[0], edge_index[1])
    mx = jnp.maximum(mx_p[0], mx_p[1])
    p = jax.nn.relu(xp + mx @ W_neigh_p + b_p)

    def bn(h, gamma, beta):
        mu = h.mean(axis=0)
        var = h.var(axis=0)
        return (h - mu) / jnp.sqrt(var + 1e-5) * gamma + beta

    h1 = jax.nn.relu(bn(m, gamma_m, beta_m) + bn(p, gamma_p, beta_p))
    sumh_p, _ = _seg_sum_sc(h1, src3d, dst3d, _D, False)
    aggh = (sumh_p[0] + sumh_p[1]) / deg[:, None]
    out = h1 @ W_self_o + aggh @ W_neigh_o + b_o
    return out


# speedup vs baseline: 4.4550x; 4.4550x over previous
import dataclasses
import functools

import jax
import jax.numpy as jnp
from jax import lax
from jax.experimental import pallas as pl
from jax.experimental.pallas import tpu as pltpu
from jax.experimental.pallas import tpu_sc as plsc

_N = 10000
_E = 320000
_D = 128
_C = 16
_BLK = 1000

_NC = 2
_NS = 16
_NW = _NC * _NS
_EPT = _E // _NW
_CH = 125
_NCH = _EPT // _CH
_STRIPE = 632
_STRIPE_LAST = _N - _STRIPE * (_NS - 1)


def _seg_sum_body(width, with_deg, table_hbm, src_hbm, dst_hbm, zeros_hbm,
                  out_hbm, deg_hbm, src_v, dst_v, rows_v, ones_v, zdeg_v,
                  acc_sh, deg_sh):
    cid = lax.axis_index("c")
    sid = lax.axis_index("s")
    wid = sid * _NC + cid

    base = sid * _STRIPE

    @pl.when(sid < _NS - 1)
    def _():
        pltpu.sync_copy(zeros_hbm.at[pl.ds(base, _STRIPE)],
                        acc_sh.at[pl.ds(base, _STRIPE)])

    @pl.when(sid == _NS - 1)
    def _():
        pltpu.sync_copy(zeros_hbm.at[pl.ds(base, _STRIPE_LAST)],
                        acc_sh.at[pl.ds(base, _STRIPE_LAST)])

    if with_deg:
        @pl.when(sid == 0)
        def _():
            @pl.loop(0, _N // 16)
            def _(i):
                zdeg_v[pl.ds(i * 16, 16)] = jnp.zeros((16,), jnp.float32)
            pltpu.sync_copy(zdeg_v, deg_sh)
        for k in range(8):
            ones_v[pl.ds(k * 16, 16)] = jnp.ones((16,), jnp.float32)

    plsc.subcore_barrier()

    pltpu.sync_copy(src_hbm.at[wid], src_v)
    pltpu.sync_copy(dst_hbm.at[wid], dst_v)

    @pl.loop(0, _NCH)
    def _(j):
        pltpu.sync_copy(table_hbm.at[src_v.at[j]], rows_v)
        pltpu.sync_copy(rows_v, acc_sh.at[dst_v.at[j]], add=True)
        if with_deg:
            pltpu.sync_copy(ones_v.at[pl.ds(0, _CH)],
                            deg_sh.at[dst_v.at[j]], add=True)

    plsc.subcore_barrier()

    @pl.when(sid < _NS - 1)
    def _():
        pltpu.sync_copy(acc_sh.at[pl.ds(base, _STRIPE)],
                        out_hbm.at[cid, pl.ds(base, _STRIPE)])

    @pl.when(sid == _NS - 1)
    def _():
        pltpu.sync_copy(acc_sh.at[pl.ds(base, _STRIPE_LAST)],
                        out_hbm.at[cid, pl.ds(base, _STRIPE_LAST)])
    if with_deg:
        @pl.when(sid == 0)
        def _():
            pltpu.sync_copy(deg_sh, deg_hbm.at[cid])


def _seg_sum_sc(table, src3d, dst3d, width, with_deg):
    mesh = plsc.VectorSubcoreMesh(core_axis_name="c", subcore_axis_name="s")
    out_type = [jax.ShapeDtypeStruct((_NC, _N, width), jnp.float32)]
    if with_deg:
        out_type.append(jax.ShapeDtypeStruct((_NC, _N), jnp.float32))
    scratch = [
        pltpu.VMEM((_NCH, _CH), jnp.int32),
        pltpu.VMEM((_NCH, _CH), jnp.int32),
        pltpu.VMEM((_CH, width), jnp.float32),
        pltpu.VMEM((_CH + 3,), jnp.float32),
        pltpu.VMEM((_N,), jnp.float32),
        pltpu.VMEM_SHARED((_N, width), jnp.float32),
        pltpu.VMEM_SHARED((_N,), jnp.float32),
    ]
    zeros = jnp.zeros((_N, width), jnp.float32)
    body = functools.partial(_seg_sum_body, width, with_deg)
    if not with_deg:
        def body2(table_hbm, src_hbm, dst_hbm, zeros_hbm, out_hbm, *rest):
            return functools.partial(_seg_sum_body, width, False)(
                table_hbm, src_hbm, dst_hbm, zeros_hbm, out_hbm, None, *rest)
        fn = pl.kernel(body2, out_type=out_type, mesh=mesh,
                       scratch_types=scratch)
        return fn(table, src3d, dst3d, zeros)[0], None
    fn = pl.kernel(body, out_type=out_type, mesh=mesh, scratch_types=scratch)
    outs = fn(table, src3d, dst3d, zeros)
    return outs[0], outs[1]


_MROWS = 632
_MROWS_LAST = _N - _MROWS * (_NS - 1)
_EH = _E // _NC
_SCH = 2000
_NSCH = _EH // _SCH
_GCH = 128
_MCAP = _SCH + 2 * _GCH
_DUMP = _MROWS


def _seg_max_body(hp_hbm, src_hbm, dst_hbm, zeros_hbm, out_hbm,
                  sbuf, dbuf, msrc, mdst, gidx, rows_v, acc_v):
    cid = lax.axis_index("c")
    sid = lax.axis_index("s")
    base = sid * _MROWS
    size = jnp.where(sid < _NS - 1, _MROWS, _MROWS_LAST)

    @pl.when(sid < _NS - 1)
    def _():
        pltpu.sync_copy(zeros_hbm.at[pl.ds(base, _MROWS)],
                        acc_v.at[pl.ds(0, _MROWS)])

    @pl.when(sid == _NS - 1)
    def _():
        pltpu.sync_copy(zeros_hbm.at[pl.ds(base, _MROWS_LAST)],
                        acc_v.at[pl.ds(0, _MROWS_LAST)])

    ebase = cid * _EH
    zeros16i = jnp.zeros((16,), jnp.int32)
    dump16i = jnp.full((16,), _DUMP, jnp.int32)
    full_mask = jnp.ones((16,), jnp.bool_)

    @pl.loop(0, _NSCH)
    def _(ci):
        off = ebase + ci * _SCH
        pltpu.sync_copy(src_hbm.at[pl.ds(off, _SCH)], sbuf.at[pl.ds(0, _SCH)])
        pltpu.sync_copy(dst_hbm.at[pl.ds(off, _SCH)], dbuf.at[pl.ds(0, _SCH)])

        def scan_g(g, ptr):
            d16 = dbuf[pl.ds(g * 16, 16)]
            t16 = d16 - base
            mask = (d16 >= base) & (t16 < size)
            s16 = sbuf[pl.ds(g * 16, 16)]
            plsc.store_compressed(msrc.at[pl.ds(ptr, 16)], s16, mask=mask)
            plsc.store_compressed(mdst.at[pl.ds(ptr, 16)], t16, mask=mask)
            pc = plsc.all_reduce_population_count(mask)
            pc = pc if pc.ndim == 0 else pc[0]
            return ptr + pc

        nm = lax.fori_loop(0, _SCH // 16, scan_g, jnp.int32(0))

        iota16 = lax.iota(jnp.int32, 16)
        for k in range(_GCH // 16):
            plsc.store_compressed(msrc.at[pl.ds(nm + k * 16, 16)],
                                  iota16 + (k * 16), mask=full_mask)
            plsc.store_compressed(mdst.at[pl.ds(nm + k * 16, 16)],
                                  dump16i, mask=full_mask)

        @pl.loop(0, _SCH // _GCH)
        def _(g):
            @pl.when(g * _GCH < nm)
            def _():
                goff = pl.multiple_of(g * _GCH, _GCH)
                for k in range(_GCH // 16):
                    gidx[0, pl.ds(k * 16, 16)] = msrc[pl.ds(goff + k * 16, 16)]
                pltpu.sync_copy(hp_hbm.at[gidx.at[0]], rows_v)

                def accum_grp(gg, _):
                    joff = pl.multiple_of(g * _GCH + gg * 16, 16)
                    ld16 = mdst[pl.ds(joff, 16)]
                    for l in range(16):
                        ld = ld16[l]
                        j = gg * 16 + l
                        for k in range(8):
                            sl = pl.ds(k * 16, 16)
                            acc_v[ld, sl] = jnp.maximum(acc_v[ld, sl],
                                                        rows_v[j, sl])
                    return 0

                lax.fori_loop(0, _GCH // 16, accum_grp, 0)

    @pl.when(sid < _NS - 1)
    def _():
        pltpu.sync_copy(acc_v.at[pl.ds(0, _MROWS)],
                        out_hbm.at[cid, pl.ds(base, _MROWS)])

    @pl.when(sid == _NS - 1)
    def _():
        pltpu.sync_copy(acc_v.at[pl.ds(0, _MROWS_LAST)],
                        out_hbm.at[cid, pl.ds(base, _MROWS_LAST)])


def _seg_max_sc(hp, src, dst):
    mesh = plsc.VectorSubcoreMesh(core_axis_name="c", subcore_axis_name="s")
    scratch = [
        pltpu.VMEM((_SCH,), jnp.int32),
        pltpu.VMEM((_SCH,), jnp.int32),
        pltpu.VMEM((_MCAP,), jnp.int32),
        pltpu.VMEM((_MCAP,), jnp.int32),
        pltpu.VMEM((1, _GCH), jnp.int32),
        pltpu.VMEM((_GCH, _D), jnp.float32),
        pltpu.VMEM((_MROWS + 8, _D), jnp.float32),
    ]
    zeros = jnp.zeros((_N, _D), jnp.float32)
    cp = pltpu.CompilerParams()
    if "needs_layout_passes" in pltpu.CompilerParams.__dataclass_fields__:
        cp = dataclasses.replace(cp, needs_layout_passes=False)
    fn = pl.kernel(_seg_max_body,
                   out_type=jax.ShapeDtypeStruct((_NC, _N, _D), jnp.float32),
                   mesh=mesh, scratch_types=scratch, compiler_params=cp)
    return fn(hp, src, dst, zeros)


def _dense_a_body(x_ref, wp_ref, bp_ref, wm_ref, wq_ref, hp_ref, xm_ref, xp_ref):
    xb = x_ref[...]
    hp_ref[...] = jnp.maximum(
        jnp.dot(xb, wp_ref[...], preferred_element_type=jnp.float32)
        + bp_ref[...], 0.0)
    xm_ref[...] = jnp.dot(xb, wm_ref[...], preferred_element_type=jnp.float32)
    xp_ref[...] = jnp.dot(xb, wq_ref[...], preferred_element_type=jnp.float32)


def _dense_a(x, W_pool, b_pool, W_self_m, W_self_p):
    n = x.shape[0]
    grid = (n // _BLK,)
    blk = pl.BlockSpec((_BLK, _D), lambda i: (i, 0))
    wspec = pl.BlockSpec((_D, _D), lambda i: (0, 0))
    bspec = pl.BlockSpec((1, _D), lambda i: (0, 0))
    out_sd = jax.ShapeDtypeStruct((n, _D), jnp.float32)
    return pl.pallas_call(
        _dense_a_body,
        grid=grid,
        in_specs=[blk, wspec, bspec, wspec, wspec],
        out_specs=[blk, blk, blk],
        out_shape=[out_sd, out_sd, out_sd],
    )(x, W_pool, b_pool.reshape(1, _D), W_self_m, W_self_p)


def kernel(x, edge_index, W_self_m, W_neigh_m, b_m, W_pool, b_pool,
           W_self_p, W_neigh_p, b_p, gamma_m, beta_m, gamma_p, beta_p,
           W_self_o, W_neigh_o, b_o):
    src3d = edge_index[0].reshape(_NW, _NCH, _CH)
    dst3d = edge_index[1].reshape(_NW, _NCH, _CH)

    hp, xm, xp = _dense_a(x, W_pool, b_pool, W_self_m, W_self_p)

    sumx_p, deg_p = _seg_sum_sc(x, src3d, dst3d, _D, True)
    deg = jnp.maximum(deg_p[0] + deg_p[1], 1.0)
    aggx = (sumx_p[0] + sumx_p[1]) / deg[:, None]
    m = jax.nn.relu(xm + aggx @ W_neigh_m + b_m)

    mx_p = _seg_max_sc(hp, edge_index[0], edge_index[1])
    mx = jnp.maximum(mx_p[0], mx_p[1])
    p = jax.nn.relu(xp + mx @ W_neigh_p + b_p)

    def bn(h, gamma, beta):
        mu = h.mean(axis=0)
        var = h.var(axis=0)
        return (h - mu) / jnp.sqrt(var + 1e-5) * gamma + beta

    h1 = jax.nn.relu(bn(m, gamma_m, beta_m) + bn(p, gamma_p, beta_p))
    sumh_p, _ = _seg_sum_sc(h1, src3d, dst3d, _D, False)
    aggh = (sumh_p[0] + sumh_p[1]) / deg[:, None]
    out = h1 @ W_self_o + aggh @ W_neigh_o + b_o
    return out
